# baseline (speedup 1.0000x reference)
import functools

import jax
import jax.numpy as jnp
from jax import lax
from jax.experimental import pallas as pl
from jax.experimental.pallas import tpu as pltpu
from jax.experimental.pallas import tpu_sc as plsc

_NC, _NS = 2, 16
_NW = _NC * _NS

_VOCAB = 1000000
_DIM = 64
_PDIM = 128
_BATCH = 4096
_HIST = 200
_B_FLAT = _BATCH * _HIST

_NBUF = 4
_NBLK = _VOCAB // _PDIM


def _relayout(tT):
    mesh = plsc.VectorSubcoreMesh(core_axis_name="c", subcore_axis_name="s")

    @functools.partial(
        pl.kernel,
        out_type=jax.ShapeDtypeStruct((_VOCAB, _PDIM), jnp.float32),
        mesh=mesh,
        scratch_types=[
            pltpu.VMEM((_DIM, _PDIM + 1), jnp.float32),
            pltpu.VMEM((_DIM, _PDIM + 1), jnp.float32),
            pltpu.VMEM((_PDIM, _PDIM), jnp.float32),
            pltpu.VMEM((_PDIM, _PDIM), jnp.float32),
            pltpu.SemaphoreType.DMA,
            pltpu.SemaphoreType.DMA,
            pltpu.SemaphoreType.DMA,
            pltpu.SemaphoreType.DMA,
        ],
        compiler_params=pltpu.CompilerParams(needs_layout_passes=False),
    )
    def k0(tT_hbm, out_hbm, ib0, ib1, ob0, ob1, gi0, gi1, go0, go1):
        wid = lax.axis_index("s") * _NC + lax.axis_index("c")
        nblk = (_NBLK - wid + _NW - 1) // _NW
        iota = lax.iota(jnp.int32, 16)
        ibufs, obufs = (ib0, ib1), (ob0, ob1)
        gis, gos = (gi0, gi1), (go0, go1)

        def stage_in(t, b):
            ct = wid + t * _NW
            col0 = pl.multiple_of(ct * _PDIM, _PDIM)
            for et in range(_DIM // 8):
                pltpu.async_copy(
                    tT_hbm.at[pl.ds(et * 8, 8), pl.ds(col0, _PDIM)],
                    ibufs[b].at[pl.ds(et * 8, 8), pl.ds(0, _PDIM)],
                    gis[b],
                )

        def wait_in(b):
            for et in range(_DIM // 8):
                pltpu.make_async_copy(
                    tT_hbm.at[pl.ds(0, 8), pl.ds(0, _PDIM)],
                    ibufs[b].at[pl.ds(et * 8, 8), pl.ds(0, _PDIM)],
                    gis[b],
                ).wait()

        def stage_out(t, b):
            ct = wid + t * _NW
            col0 = pl.multiple_of(ct * _PDIM, _PDIM)
            return pltpu.async_copy(obufs[b], out_hbm.at[pl.ds(col0, _PDIM)], gos[b])

        stage_in(0, 0)
        n_pairs = (nblk + 1) // 2

        def pair(r, carry):
            for b in range(2):
                t = 2 * r + b

                @pl.when(t < nblk)
                def _():
                    wait_in(b)

                    @pl.when(t + 1 < nblk)
                    def _():
                        stage_in(t + 1, 1 - b)

                    def row(i, c2):
                        for kk in range(_DIM // 16):
                            rv = iota + (16 * kk)
                            cv = jnp.full((16,), i, jnp.int32)
                            vals = plsc.load_gather(ibufs[b], [rv, cv])
                            obufs[b][i, pl.ds(16 * kk, 16)] = vals
                        return c2

                    lax.fori_loop(0, _PDIM, row, 0, unroll=8)
                    stage_out(t, b).wait()

            return carry

        lax.fori_loop(0, n_pairs, pair, 0)

    return k0(tT)


def _gather(idx2_flat, table2):
    b_per_w = _B_FLAT // _NW
    batches_per_w = _BATCH // _NW
    n_chunks = batches_per_w
    n_rounds = n_chunks // _NBUF
    mesh = plsc.VectorSubcoreMesh(core_axis_name="c", subcore_axis_name="s")

    scratch = [pltpu.VMEM((b_per_w,), jnp.int32)]
    scratch += [pltpu.VMEM((_HIST, _DIM), jnp.float32) for _ in range(_NBUF)]
    scratch += [pltpu.SemaphoreType.DMA for _ in range(2 * _NBUF)]

    @functools.partial(
        pl.kernel,
        out_type=jax.ShapeDtypeStruct((_BATCH, _HIST, _PDIM), jnp.float32),
        mesh=mesh,
        scratch_types=scratch,
        compiler_params=pltpu.CompilerParams(use_tc_tiling_on_sc=False),
    )
    def gather_kernel(idx_hbm, table_hbm, out_hbm, idx_v, *bufs_and_sems):
        bufs = bufs_and_sems[:_NBUF]
        gsems = bufs_and_sems[_NBUF : 2 * _NBUF]
        osems = bufs_and_sems[2 * _NBUF :]

        wid = lax.axis_index("s") * _NC + lax.axis_index("c")
        base = wid * b_per_w
        bbase = wid * batches_per_w
        pltpu.sync_copy(idx_hbm.at[pl.ds(base, b_per_w)], idx_v)

        def gather_start(c, k):
            return pltpu.async_copy(
                table_hbm.at[idx_v.at[pl.ds(c * _HIST, _HIST)]], bufs[k], gsems[k]
            )

        def gather_wait(k):
            pltpu.make_async_copy(
                table_hbm.at[idx_v.at[pl.ds(0, _HIST)]], bufs[k], gsems[k]
            ).wait()

        for k in range(_NBUF):
            gather_start(k, k)

        def round_body(r, carry):
            for k in range(_NBUF):
                c = r * _NBUF + k
                gather_wait(k)
                pltpu.async_copy(
                    bufs[k],
                    out_hbm.at[bbase + c].at[:, pl.ds(0, _DIM)],
                    osems[k],
                ).wait()

                @pl.when(c + _NBUF < n_chunks)
                def _():
                    gather_start(c + _NBUF, k)

            return carry

        lax.fori_loop(0, n_rounds, round_body, 0)

    return gather_kernel(idx2_flat, table2)


@jax.jit
def _run(batchinput, table):
    idx2_flat = batchinput.reshape(-1).astype(jnp.int32) * 2
    t128 = _relayout(table.T)
    tail = jnp.pad(
        table[_NBLK * _PDIM :], ((0, 0), (0, _PDIM - _DIM))
    )
    t128 = lax.dynamic_update_slice(t128, tail, (_NBLK * _PDIM, 0))
    table2 = t128.reshape(2 * _VOCAB, _DIM)
    out128 = _gather(idx2_flat, table2)
    return out128[:, :, :_DIM]


def kernel(batchinput, table):
    return _run(batchinput, table)


# speedup vs baseline: 2.2952x; 2.2952x over previous
import functools

import jax
import jax.numpy as jnp
from jax import lax
from jax.experimental import pallas as pl
from jax.experimental.pallas import tpu as pltpu
from jax.experimental.pallas import tpu_sc as plsc

_NC, _NS = 2, 16
_NW = _NC * _NS

_VOCAB = 1000000
_DIM = 64
_PDIM = 128
_BATCH = 4096
_HIST = 200
_B_FLAT = _BATCH * _HIST

_NBUF = 4


@jax.jit
def _gather(idx2_flat, table2):
    b_per_w = _B_FLAT // _NW
    batches_per_w = _BATCH // _NW
    n_chunks = batches_per_w
    n_rounds = n_chunks // _NBUF
    mesh = plsc.VectorSubcoreMesh(core_axis_name="c", subcore_axis_name="s")

    scratch = [pltpu.VMEM((b_per_w,), jnp.int32)]
    scratch += [pltpu.VMEM((_HIST, _DIM), jnp.float32) for _ in range(_NBUF)]
    scratch += [pltpu.SemaphoreType.DMA for _ in range(2 * _NBUF)]

    @functools.partial(
        pl.kernel,
        out_type=jax.ShapeDtypeStruct((_BATCH, _HIST, _PDIM), jnp.float32),
        mesh=mesh,
        scratch_types=scratch,
        compiler_params=pltpu.CompilerParams(use_tc_tiling_on_sc=False),
    )
    def gather_kernel(idx_hbm, table_hbm, out_hbm, idx_v, *bufs_and_sems):
        bufs = bufs_and_sems[:_NBUF]
        gsems = bufs_and_sems[_NBUF : 2 * _NBUF]
        osems = bufs_and_sems[2 * _NBUF :]

        wid = lax.axis_index("s") * _NC + lax.axis_index("c")
        base = wid * b_per_w
        bbase = wid * batches_per_w
        pltpu.sync_copy(idx_hbm.at[pl.ds(base, b_per_w)], idx_v)

        def gather_start(c, k):
            return pltpu.async_copy(
                table_hbm.at[idx_v.at[pl.ds(c * _HIST, _HIST)]], bufs[k], gsems[k]
            )

        def gather_wait(k):
            pltpu.make_async_copy(
                table_hbm.at[idx_v.at[pl.ds(0, _HIST)]], bufs[k], gsems[k]
            ).wait()

        for k in range(_NBUF):
            gather_start(k, k)

        def round_body(r, carry):
            for k in range(_NBUF):
                c = r * _NBUF + k
                gather_wait(k)
                pltpu.async_copy(
                    bufs[k],
                    out_hbm.at[bbase + c].at[:, pl.ds(0, _DIM)],
                    osems[k],
                ).wait()

                @pl.when(c + _NBUF < n_chunks)
                def _():
                    gather_start(c + _NBUF, k)

            return carry

        lax.fori_loop(0, n_rounds, round_body, 0)

    return gather_kernel(idx2_flat, table2)


def kernel(batchinput, table):
    idx2_flat = batchinput.reshape(-1).astype(jnp.int32) * 2
    table2 = jnp.pad(table, ((0, 0), (0, _PDIM - _DIM))).reshape(2 * _VOCAB, _DIM)
    out128 = _gather(idx2_flat, table2)
    return out128[:, :, :_DIM]
